# Initial kernel scaffold; baseline (speedup 1.0000x reference)
import jax
import jax.numpy as jnp
from jax.experimental import pallas as pl


def kernel(input, mask, stride, triple_w, flag, show_flow):
    raise NotImplementedError("write your pallas kernel here")


# speedup vs baseline: 1.4225x; 1.4225x over previous
import functools

import jax
import jax.numpy as jnp
from jax.experimental import pallas as pl
from jax.experimental.pallas import tpu as pltpu

_BLK = 512


def _shift_body(flag_ref, flagc_ref, f_ref, l_ref, o_ref):
    F = f_ref[0]
    L = l_ref[0]
    colflag = flag_ref[:, :]
    rowflag = flagc_ref[:, :]

    c2 = F.shape[0]
    ones_c = jnp.ones((c2, 1), dtype=jnp.float32)
    nf = jnp.sqrt(jax.lax.dot_general(F * F, ones_c, (((0,), (0,)), ((), ())),
                                      preferred_element_type=jnp.float32))
    nl = jnp.sqrt(jnp.sum(L * L, axis=0, keepdims=True))

    S = jax.lax.dot_general(F, L, (((0,), (0,)), ((), ())),
                            preferred_element_type=jnp.float32)
    denom = nf * nl + jnp.float32(1e-8)
    cos = S / denom

    colok = colflag == 0
    cm = jnp.where(colok, cos, -jnp.inf)
    m = jnp.max(cm, axis=1, keepdims=True)
    e = jnp.exp(cm - m)
    s = jnp.sum(e, axis=1, keepdims=True)
    p = e / s
    rowok = rowflag == 1
    T = jnp.where(rowok & colok, p, jnp.float32(0.0))

    O = jax.lax.dot_general(L, T, (((1,), (1,)), ((), ())),
                            preferred_element_type=jnp.float32)
    o_ref[0] = O


@jax.jit
def _shift(inp, flag):
    bz, c, h, w = inp.shape
    c2 = c // 2
    HW = h * w
    x = inp.reshape(bz, c, HW)
    former = x[:, :c2]
    latter = x[:, c2:]
    flag2 = flag.astype(jnp.int32).reshape(1, HW)
    flagc = flag.astype(jnp.int32).reshape(HW, 1)

    grid = (bz, HW // _BLK)
    shifted = pl.pallas_call(
        _shift_body,
        grid=grid,
        in_specs=[
            pl.BlockSpec((1, HW), lambda b, i: (0, 0)),
            pl.BlockSpec((_BLK, 1), lambda b, i: (i, 0)),
            pl.BlockSpec((1, c2, _BLK), lambda b, i: (b, 0, i)),
            pl.BlockSpec((1, c2, HW), lambda b, i: (b, 0, 0)),
        ],
        out_specs=pl.BlockSpec((1, c2, _BLK), lambda b, i: (b, 0, i)),
        out_shape=jax.ShapeDtypeStruct((bz, c2, HW), jnp.float32),
        compiler_params=pltpu.CompilerParams(
            dimension_semantics=("parallel", "arbitrary"),
        ),
    )(flag2, flagc, former, latter)
    return jnp.concatenate([inp, shifted.reshape(bz, c2, h, w)], axis=1)


def kernel(input, mask, stride, triple_w, flag, show_flow):
    return _shift(input, flag)


# speedup vs baseline: 1.7988x; 1.2645x over previous
import jax
import jax.numpy as jnp
from jax.experimental import pallas as pl
from jax.experimental.pallas import tpu as pltpu

_BLK = 512


def _shift_body(flag_ref, flagc_ref, f_ref, l_ref, o_ref):
    F = f_ref[0]
    L = l_ref[0]
    colflag = flag_ref[:, :]
    rowflag = flagc_ref[:, :]

    inv_nf = jax.lax.rsqrt(jnp.maximum(
        jnp.sum(F * F, axis=0, keepdims=True), jnp.float32(1e-16)))
    inv_nl = jax.lax.rsqrt(jnp.maximum(
        jnp.sum(L * L, axis=0, keepdims=True), jnp.float32(1e-16)))
    Fh = (F * inv_nf).astype(jnp.bfloat16)
    Lh = (L * inv_nl).astype(jnp.bfloat16)

    cos = jax.lax.dot_general(Fh, Lh, (((0,), (0,)), ((), ())),
                              preferred_element_type=jnp.float32
                              ).astype(jnp.bfloat16)

    colok = colflag == 0
    rowok = rowflag == 1
    cm = jnp.where(colok, cos, -jnp.inf)
    e = jnp.exp(cm)
    T = jnp.where(rowok & colok, e, jnp.bfloat16(0.0))

    Lb = L.astype(jnp.bfloat16)
    ones_row = jnp.ones((1, Lb.shape[1]), dtype=jnp.bfloat16)
    Laug = jnp.concatenate([Lb, ones_row], axis=0)
    O = jax.lax.dot_general(Laug, T, (((1,), (1,)), ((), ())),
                            preferred_element_type=jnp.float32)
    s = jnp.maximum(O[-1:, :], jnp.float32(1e-30))
    o_ref[0] = O[:-1, :] * (jnp.float32(1.0) / s)


@jax.jit
def _shift(inp, flag):
    bz, c, h, w = inp.shape
    c2 = c // 2
    HW = h * w
    x = inp.reshape(bz, c, HW)
    former = x[:, :c2]
    latter = x[:, c2:]
    flag2 = flag.astype(jnp.int32).reshape(1, HW)
    flagc = flag.astype(jnp.int32).reshape(HW, 1)

    grid = (bz, HW // _BLK)
    shifted = pl.pallas_call(
        _shift_body,
        grid=grid,
        in_specs=[
            pl.BlockSpec((1, HW), lambda b, i: (0, 0)),
            pl.BlockSpec((_BLK, 1), lambda b, i: (i, 0)),
            pl.BlockSpec((1, c2, _BLK), lambda b, i: (b, 0, i)),
            pl.BlockSpec((1, c2, HW), lambda b, i: (b, 0, 0)),
        ],
        out_specs=pl.BlockSpec((1, c2, _BLK), lambda b, i: (b, 0, i)),
        out_shape=jax.ShapeDtypeStruct((bz, c2, HW), jnp.float32),
        compiler_params=pltpu.CompilerParams(
            dimension_semantics=("parallel", "arbitrary"),
        ),
    )(flag2, flagc, former, latter)
    return jnp.concatenate([inp, shifted.reshape(bz, c2, h, w)], axis=1)


def kernel(input, mask, stride, triple_w, flag, show_flow):
    return _shift(input, flag)


# speedup vs baseline: 1.8711x; 1.0402x over previous
import jax
import jax.numpy as jnp
from jax.experimental import pallas as pl
from jax.experimental.pallas import tpu as pltpu

_BLK = 512


def _shift_body(flag_ref, flagc_ref, f_ref, l_ref, o_ref, lh_s, laug_s):
    i = pl.program_id(1)
    c2 = f_ref.shape[1]
    HW = l_ref.shape[2]

    @pl.when(i == 0)
    def _prep_latter():
        L = l_ref[0]
        inv_nl = jax.lax.rsqrt(jnp.maximum(
            jnp.sum(L * L, axis=0, keepdims=True), jnp.float32(1e-16)))
        lh_s[...] = (L * inv_nl).astype(jnp.bfloat16)
        laug_s[0:c2, :] = L.astype(jnp.bfloat16)
        laug_s[c2:c2 + 8, :] = jnp.ones((8, HW), dtype=jnp.bfloat16)

    F = f_ref[0]
    colflag = flag_ref[:, :]
    rowflag = flagc_ref[:, :]

    inv_nf = jax.lax.rsqrt(jnp.maximum(
        jnp.sum(F * F, axis=0, keepdims=True), jnp.float32(1e-16)))
    Fh = (F * inv_nf).astype(jnp.bfloat16)

    cos = jax.lax.dot_general(Fh, lh_s[...], (((0,), (0,)), ((), ())),
                              preferred_element_type=jnp.float32
                              ).astype(jnp.bfloat16)

    colok = colflag == 0
    rowok = rowflag == 1
    cm = jnp.where(colok, cos, -jnp.inf)
    e = jnp.exp(cm)
    T = jnp.where(rowok & colok, e, jnp.bfloat16(0.0))

    O = jax.lax.dot_general(laug_s[...], T, (((1,), (1,)), ((), ())),
                            preferred_element_type=jnp.float32)
    s = jnp.maximum(O[c2:c2 + 1, :], jnp.float32(1e-30))
    o_ref[0] = O[0:c2, :] * (jnp.float32(1.0) / s)


@jax.jit
def _shift(inp, flag):
    bz, c, h, w = inp.shape
    c2 = c // 2
    HW = h * w
    x = inp.reshape(bz, c, HW)
    former = x[:, :c2]
    latter = x[:, c2:]
    flag2 = flag.astype(jnp.int32).reshape(1, HW)
    flagc = flag.astype(jnp.int32).reshape(HW, 1)

    grid = (bz, HW // _BLK)
    shifted = pl.pallas_call(
        _shift_body,
        grid=grid,
        in_specs=[
            pl.BlockSpec((1, HW), lambda b, i: (0, 0)),
            pl.BlockSpec((_BLK, 1), lambda b, i: (i, 0)),
            pl.BlockSpec((1, c2, _BLK), lambda b, i: (b, 0, i)),
            pl.BlockSpec((1, c2, HW), lambda b, i: (b, 0, 0)),
        ],
        out_specs=pl.BlockSpec((1, c2, _BLK), lambda b, i: (b, 0, i)),
        out_shape=jax.ShapeDtypeStruct((bz, c2, HW), jnp.float32),
        scratch_shapes=[
            pltpu.VMEM((c2, HW), jnp.bfloat16),
            pltpu.VMEM((c2 + 8, HW), jnp.bfloat16),
        ],
        compiler_params=pltpu.CompilerParams(
            dimension_semantics=("parallel", "arbitrary"),
        ),
    )(flag2, flagc, former, latter)
    return jnp.concatenate([inp, shifted.reshape(bz, c2, h, w)], axis=1)


def kernel(input, mask, stride, triple_w, flag, show_flow):
    return _shift(input, flag)


# speedup vs baseline: 2.1661x; 1.1577x over previous
import jax
import jax.numpy as jnp
from jax.experimental import pallas as pl
from jax.experimental.pallas import tpu as pltpu

_BLK = 512


def _shift_body(flag_ref, f_ref, l_ref, o_ref, lh_s, laug_s):
    i = pl.program_id(1)
    c2 = f_ref.shape[1]
    HW = l_ref.shape[2]

    @pl.when(i == 0)
    def _prep_latter():
        L = l_ref[0]
        inv_nl = jax.lax.rsqrt(jnp.maximum(
            jnp.sum(L * L, axis=0, keepdims=True), jnp.float32(1e-16)))
        lh_s[...] = (L * inv_nl).astype(jnp.bfloat16)
        colmask = (flag_ref[:, :] == 0).astype(jnp.float32)
        laug_s[0:c2, :] = (L * colmask).astype(jnp.bfloat16)
        laug_s[c2:c2 + 8, :] = jnp.broadcast_to(
            colmask.astype(jnp.bfloat16), (8, HW))

    F = f_ref[0]
    inv_nf = jax.lax.rsqrt(jnp.maximum(
        jnp.sum(F * F, axis=0, keepdims=True), jnp.float32(1e-16)))
    Fh = (F * inv_nf).astype(jnp.bfloat16)

    cosT = jax.lax.dot_general(lh_s[...], Fh, (((0,), (0,)), ((), ())),
                               preferred_element_type=jnp.float32)
    e = jnp.exp(cosT.astype(jnp.bfloat16))

    O = jax.lax.dot_general(laug_s[...], e, (((1,), (0,)), ((), ())),
                            preferred_element_type=jnp.float32)
    s = jnp.maximum(O[c2:c2 + 1, :], jnp.float32(1e-30))
    rowmask = (flag_ref[:, pl.ds(i * _BLK, _BLK)] == 1).astype(jnp.float32)
    o_ref[0] = O[0:c2, :] * (rowmask / s)


@jax.jit
def _shift(inp, flag):
    bz, c, h, w = inp.shape
    c2 = c // 2
    HW = h * w
    x = inp.reshape(bz, c, HW)
    former = x[:, :c2]
    latter = x[:, c2:]
    flag2 = flag.astype(jnp.int32).reshape(1, HW)

    grid = (bz, HW // _BLK)
    shifted = pl.pallas_call(
        _shift_body,
        grid=grid,
        in_specs=[
            pl.BlockSpec((1, HW), lambda b, i: (0, 0)),
            pl.BlockSpec((1, c2, _BLK), lambda b, i: (b, 0, i)),
            pl.BlockSpec((1, c2, HW), lambda b, i: (b, 0, 0)),
        ],
        out_specs=pl.BlockSpec((1, c2, _BLK), lambda b, i: (b, 0, i)),
        out_shape=jax.ShapeDtypeStruct((bz, c2, HW), jnp.float32),
        scratch_shapes=[
            pltpu.VMEM((c2, HW), jnp.bfloat16),
            pltpu.VMEM((c2 + 8, HW), jnp.bfloat16),
        ],
        compiler_params=pltpu.CompilerParams(
            dimension_semantics=("parallel", "arbitrary"),
        ),
    )(flag2, former, latter)
    return jnp.concatenate([inp, shifted.reshape(bz, c2, h, w)], axis=1)


def kernel(input, mask, stride, triple_w, flag, show_flow):
    return _shift(input, flag)


# speedup vs baseline: 2.5117x; 1.1595x over previous
import jax
import jax.numpy as jnp
from jax.experimental import pallas as pl
from jax.experimental.pallas import tpu as pltpu

_BLK = 512


def _shift_body(flag_ref, x_ref, lat_ref, o_ref, lh_s, laug_s):
    i = pl.program_id(1)
    c = x_ref.shape[1]
    c2 = c // 2
    HW = lat_ref.shape[2]

    @pl.when(i == 0)
    def _prep_latter():
        L = lat_ref[0]
        inv_nl = jax.lax.rsqrt(jnp.maximum(
            jnp.sum(L * L, axis=0, keepdims=True), jnp.float32(1e-16)))
        lh_s[...] = (L * inv_nl).astype(jnp.bfloat16)
        colmask = (flag_ref[:, :] == 0).astype(jnp.float32)
        laug_s[0:c2, :] = (L * colmask).astype(jnp.bfloat16)
        laug_s[c2:c2 + 8, :] = jnp.broadcast_to(
            colmask.astype(jnp.bfloat16), (8, HW))

    X = x_ref[0]
    o_ref[0, 0:c, :] = X

    F = X[0:c2, :]
    inv_nf = jax.lax.rsqrt(jnp.maximum(
        jnp.sum(F * F, axis=0, keepdims=True), jnp.float32(1e-16)))
    Fh = (F * inv_nf).astype(jnp.bfloat16)

    cosT = jax.lax.dot_general(lh_s[...], Fh, (((0,), (0,)), ((), ())),
                               preferred_element_type=jnp.float32)
    e = jnp.exp(cosT.astype(jnp.bfloat16))

    O = jax.lax.dot_general(laug_s[...], e, (((1,), (0,)), ((), ())),
                            preferred_element_type=jnp.float32)
    s = jnp.maximum(O[c2:c2 + 1, :], jnp.float32(1e-30))
    rowmask = (flag_ref[:, pl.ds(i * _BLK, _BLK)] == 1).astype(jnp.float32)
    o_ref[0, c:c + c2, :] = O[0:c2, :] * (rowmask / s)


@jax.jit
def _shift(inp, flag):
    bz, c, h, w = inp.shape
    c2 = c // 2
    HW = h * w
    x = inp.reshape(bz, c, HW)
    flag2 = flag.astype(jnp.int32).reshape(1, HW)

    grid = (bz, HW // _BLK)
    out = pl.pallas_call(
        _shift_body,
        grid=grid,
        in_specs=[
            pl.BlockSpec((1, HW), lambda b, i: (0, 0)),
            pl.BlockSpec((1, c, _BLK), lambda b, i: (b, 0, i)),
            pl.BlockSpec((1, c2, HW), lambda b, i: (b, 1, 0)),
        ],
        out_specs=pl.BlockSpec((1, c + c2, _BLK), lambda b, i: (b, 0, i)),
        out_shape=jax.ShapeDtypeStruct((bz, c + c2, HW), jnp.float32),
        scratch_shapes=[
            pltpu.VMEM((c2, HW), jnp.bfloat16),
            pltpu.VMEM((c2 + 8, HW), jnp.bfloat16),
        ],
        compiler_params=pltpu.CompilerParams(
            dimension_semantics=("parallel", "arbitrary"),
        ),
    )(flag2, x, x)
    return out.reshape(bz, c + c2, h, w)


def kernel(input, mask, stride, triple_w, flag, show_flow):
    return _shift(input, flag)
